# baseline (speedup 1.0000x reference)
import functools

import numpy as np

import jax
import jax.numpy as jnp
from jax import lax
from jax.experimental import pallas as pl
from jax.experimental.pallas import tpu as pltpu
from jax.experimental.pallas import tpu_sc as plsc


def _reduce_body(x_ref, c_ref, gidx_ref, sel_ref, maskf_ref, acc_ref):
    BK = x_ref.shape[0] * x_ref.shape[1]
    B = x_ref.shape[2]
    W = x_ref.shape[1]
    xb = x_ref[...].reshape(BK, B)
    cb = c_ref[...]
    r = lax.dot_general(
        cb, xb,
        dimension_numbers=(((1,), (0,)), ((), ())),
        preferred_element_type=jnp.int32,
    )

    @pl.when(pl.program_id(0) == 0)
    def _init():
        acc_ref[...] = jnp.zeros_like(acc_ref)

    acc_ref[...] += r

    @pl.when(pl.program_id(0) == pl.num_programs(0) - 1)
    def _fini():
        s = acc_ref[...]
        count = s[0].astype(jnp.float32)
        ysum = (s[1] + 128 * s[2]).astype(jnp.float32)
        xsum = (s[3] + 128 * s[4]).astype(jnp.float32)
        safe = jnp.maximum(count, 1.0)
        nz = count > 0.0
        ty = jnp.where(nz, ysum / safe, 0.0).astype(jnp.int32)
        tx = jnp.where(nz, xsum / safe, 0.0).astype(jnp.int32)
        idx = ty * W + tx
        gidx_ref[...] = idx >> 3
        sel_ref[...] = (idx & 7)[:, None]
        maskf_ref[...] = nz.astype(jnp.float32)[:, None]


def _tc_reduce(xt, coeffs):
    H, W, B = xt.shape
    HB = 28
    grid = H // HB
    return pl.pallas_call(
        _reduce_body,
        grid=(grid,),
        in_specs=[
            pl.BlockSpec((HB, W, B), lambda i: (i, 0, 0)),
            pl.BlockSpec((5, HB * W), lambda i: (0, i)),
        ],
        out_specs=[
            pl.BlockSpec((B,), lambda i: (0,)),
            pl.BlockSpec((B, 1), lambda i: (0, 0)),
            pl.BlockSpec((B, 1), lambda i: (0, 0)),
        ],
        out_shape=[
            jax.ShapeDtypeStruct((B,), jnp.int32),
            jax.ShapeDtypeStruct((B, 1), jnp.int32),
            jax.ShapeDtypeStruct((B, 1), jnp.float32),
        ],
        scratch_shapes=[pltpu.VMEM((5, B), jnp.int32)],
    )(xt, coeffs)


def _make_coeffs(H, W):
    k = np.arange(H * W, dtype=np.int32)
    h = k // W
    w = k % W
    rows = np.stack([np.ones_like(k), h % 128, h // 128, w % 128, w // 128])
    return jnp.asarray(rows.astype(np.int8))


def _make_sc_gather(B, G, D):
    info = plsc.get_sparse_core_info()
    NC, NS = info.num_cores, info.num_subcores
    NW = NC * NS
    assert B % (8 * NW) == 0
    bpw = B // NW
    mesh = plsc.VectorSubcoreMesh(core_axis_name="c", subcore_axis_name="s")

    @functools.partial(
        pl.kernel,
        mesh=mesh,
        out_type=jax.ShapeDtypeStruct((B, G, D), jnp.float32),
        scratch_types=[
            pltpu.VMEM((bpw,), jnp.int32),
            pltpu.VMEM((bpw, G, D), jnp.float32),
            pltpu.SemaphoreType.DMA,
        ],
    )
    def sc_k(table_hbm, gidx_hbm, out_hbm, idx_v, rows_v, sem):
        wid = lax.axis_index("s") * NC + lax.axis_index("c")
        base = wid * bpw
        pltpu.sync_copy(gidx_hbm.at[pl.ds(base, bpw)], idx_v)
        pltpu.async_copy(table_hbm.at[idx_v], rows_v, sem).wait()
        pltpu.sync_copy(rows_v, out_hbm.at[pl.ds(base, bpw)])

    return sc_k


def _combine_body(rows_ref, sel_ref, maskf_ref, g_ref, e_ref, out_ref):
    G = rows_ref.shape[1]
    sel = sel_ref[...]
    m = maskf_ref[...]
    row = (sel == 0).astype(jnp.float32) * rows_ref[:, 0, :]
    for k in range(1, G):
        row += (sel == k).astype(jnp.float32) * rows_ref[:, k, :]
    gme = (g_ref[...] - e_ref[...])[None, :]
    out_ref[...] = e_ref[...][None, :] + m * (row + gme)


def _tc_combine(rows8, sel, maskf, g, e):
    B, G, D = rows8.shape
    BB = 128
    grid = B // BB
    return pl.pallas_call(
        _combine_body,
        grid=(grid,),
        in_specs=[
            pl.BlockSpec((BB, G, D), lambda i: (i, 0, 0)),
            pl.BlockSpec((BB, 1), lambda i: (i, 0)),
            pl.BlockSpec((BB, 1), lambda i: (i, 0)),
            pl.BlockSpec((D,), lambda i: (0,)),
            pl.BlockSpec((D,), lambda i: (0,)),
        ],
        out_specs=pl.BlockSpec((BB, D), lambda i: (i, 0)),
        out_shape=jax.ShapeDtypeStruct((B, D), jnp.float32),
    )(rows8, sel, maskf, g, e)


def kernel(x, pe, global_emb, empty_emb):
    B, H, W = x.shape
    D = pe.shape[-1]
    G = 8
    xt = jnp.transpose(x, (1, 2, 0)).astype(jnp.int8)
    coeffs = _make_coeffs(H, W)
    gidx, sel, maskf = _tc_reduce(xt, coeffs)
    table = pe.reshape(H * W // G, G, D)
    sc_k = _make_sc_gather(B, G, D)
    rows8 = sc_k(table, gidx)
    return _tc_combine(rows8, sel, maskf, global_emb, empty_emb)


# speedup vs baseline: 1.1207x; 1.1207x over previous
import functools

import numpy as np

import jax
import jax.numpy as jnp
from jax import lax
from jax.experimental import pallas as pl
from jax.experimental.pallas import tpu as pltpu
from jax.experimental.pallas import tpu_sc as plsc


def _reduce_body(x_ref, c_ref, idx_ref, maskf_ref, acc_ref):
    BK = x_ref.shape[0] * x_ref.shape[1]
    B = x_ref.shape[2]
    W = x_ref.shape[1]
    xb = x_ref[...].reshape(BK, B)
    cb = c_ref[...]
    r = lax.dot_general(
        cb, xb,
        dimension_numbers=(((1,), (0,)), ((), ())),
        preferred_element_type=jnp.int32,
    )

    @pl.when(pl.program_id(0) == 0)
    def _init():
        acc_ref[...] = jnp.zeros_like(acc_ref)

    acc_ref[...] += r

    @pl.when(pl.program_id(0) == pl.num_programs(0) - 1)
    def _fini():
        s = acc_ref[...]
        count = s[0].astype(jnp.float32)
        ysum = (s[1] + 128 * s[2]).astype(jnp.float32)
        xsum = (s[3] + 128 * s[4]).astype(jnp.float32)
        safe = jnp.maximum(count, 1.0)
        nz = count > 0.0
        ty = jnp.where(nz, ysum / safe, 0.0).astype(jnp.int32)
        tx = jnp.where(nz, xsum / safe, 0.0).astype(jnp.int32)
        idx_ref[...] = ty * W + tx
        maskf_ref[...] = nz.astype(jnp.float32)[:, None]


def _tc_reduce(xt, coeffs):
    H, W, B = xt.shape
    HB = 56
    grid = H // HB
    return pl.pallas_call(
        _reduce_body,
        grid=(grid,),
        in_specs=[
            pl.BlockSpec((HB, W, B), lambda i: (i, 0, 0)),
            pl.BlockSpec((5, HB * W), lambda i: (0, i)),
        ],
        out_specs=[
            pl.BlockSpec((B,), lambda i: (0,)),
            pl.BlockSpec((B, 1), lambda i: (0, 0)),
        ],
        out_shape=[
            jax.ShapeDtypeStruct((B,), jnp.int32),
            jax.ShapeDtypeStruct((B, 1), jnp.float32),
        ],
        scratch_shapes=[pltpu.VMEM((5, B), jnp.int32)],
    )(xt, coeffs)


def _make_coeffs(H, W):
    k = np.arange(H * W, dtype=np.int32)
    h = k // W
    w = k % W
    rows = np.stack([np.ones_like(k), h % 128, h // 128, w % 128, w // 128])
    return jnp.asarray(rows.astype(np.int8))


def _make_sc_gather(B, D):
    info = plsc.get_sparse_core_info()
    NC, NS = info.num_cores, info.num_subcores
    NW = NC * NS
    assert B % (8 * NW) == 0
    bpw = B // NW
    NSTREAM = 4
    chunk = bpw // NSTREAM
    mesh = plsc.VectorSubcoreMesh(core_axis_name="c", subcore_axis_name="s")

    @functools.partial(
        pl.kernel,
        mesh=mesh,
        out_type=jax.ShapeDtypeStruct((B, D), jnp.float32),
        scratch_types=[
            pltpu.VMEM((bpw,), jnp.int32),
            pltpu.VMEM((bpw, D), jnp.float32),
        ]
        + [pltpu.SemaphoreType.DMA] * NSTREAM,
    )
    def sc_k(table_hbm, idx_hbm, out_hbm, idx_v, rows_v, *sems):
        wid = lax.axis_index("s") * NC + lax.axis_index("c")
        base = wid * bpw
        pltpu.sync_copy(idx_hbm.at[pl.ds(base, bpw)], idx_v)
        copies = []
        for j in range(NSTREAM):
            copies.append(pltpu.async_copy(
                table_hbm.at[idx_v.at[pl.ds(j * chunk, chunk)]],
                rows_v.at[pl.ds(j * chunk, chunk)],
                sems[j],
            ))
        for c in copies:
            c.wait()
        pltpu.sync_copy(rows_v, out_hbm.at[pl.ds(base, bpw)])

    return sc_k


def _combine_body(rows_ref, maskf_ref, g_ref, e_ref, out_ref):
    rows = rows_ref[...]
    m = maskf_ref[...]
    gme = (g_ref[...] - e_ref[...])[None, :]
    out_ref[...] = e_ref[...][None, :] + m * (rows + gme)


def _tc_combine(rows, maskf, g, e):
    B, D = rows.shape
    BB = 256
    grid = B // BB
    return pl.pallas_call(
        _combine_body,
        grid=(grid,),
        in_specs=[
            pl.BlockSpec((BB, D), lambda i: (i, 0)),
            pl.BlockSpec((BB, 1), lambda i: (i, 0)),
            pl.BlockSpec((D,), lambda i: (0,)),
            pl.BlockSpec((D,), lambda i: (0,)),
        ],
        out_specs=pl.BlockSpec((BB, D), lambda i: (i, 0)),
        out_shape=jax.ShapeDtypeStruct((B, D), jnp.float32),
    )(rows, maskf, g, e)


def kernel(x, pe, global_emb, empty_emb):
    B, H, W = x.shape
    D = pe.shape[-1]
    xt = jnp.transpose(x, (1, 2, 0)).astype(jnp.int8)
    coeffs = _make_coeffs(H, W)
    idx, maskf = _tc_reduce(xt, coeffs)
    table = pe.reshape(H * W, D)
    sc_k = _make_sc_gather(B, D)
    rows = sc_k(table, idx)
    return _tc_combine(rows, maskf, global_emb, empty_emb)
